# baseline (speedup 1.0000x reference)
import jax
import jax.numpy as jnp
from jax import lax
from jax.experimental import pallas as pl
from jax.experimental.pallas import tpu as pltpu
from jax.experimental.pallas import tpu_sc as plsc

_WORD_NUM = 1000000
_ENTITY_NUM = 1000000
_EMBED = 32
_FACTOR = 0.5
_L2 = 1e-06
_B = 16384
_L = 20
_K = 64

_NC, _NS = 2, 16
_NW = _NC * _NS
_BC = _B // _NW
_QC = _B * _L // _NW
_QCH = 16
_QG = 32
_QROWS = _QG * _L


_CJ = 8192
_G1 = (_WORD_NUM + _CJ - 1) // _CJ


def _k1_body(wt, et, pad, nrm, acc):
    i = pl.program_id(0)

    @pl.when(i == 0)
    def _():
        acc[0] = 0.0
        acc[1] = 0.0

    limit = _WORD_NUM - i * _CJ
    col = lax.broadcasted_iota(jnp.int32, (_EMBED, _CJ), 1)
    m = col < limit
    w = jnp.where(m, wt[...], 0.0)
    e = jnp.where(m, et[...], 0.0)
    acc[0] += jnp.sum(w * w)
    acc[1] += jnp.sum(e * e)
    z = jnp.zeros((_CJ, 128 - 2 * _EMBED), jnp.float32)
    pad[...] = jnp.concatenate([jnp.transpose(w), jnp.transpose(e), z], axis=1)

    @pl.when(i == _G1 - 1)
    def _():
        nrm[0, 0] = jnp.sqrt(acc[0])
        nrm[0, 1] = jnp.sqrt(acc[1])


def _k1(wT, eT):
    return pl.pallas_call(
        _k1_body,
        grid=(_G1,),
        in_specs=[pl.BlockSpec((_EMBED, _CJ), lambda i: (0, i)),
                  pl.BlockSpec((_EMBED, _CJ), lambda i: (0, i))],
        out_specs=[pl.BlockSpec((_CJ, 128), lambda i: (i, 0)),
                   pl.BlockSpec(memory_space=pltpu.SMEM)],
        out_shape=[jax.ShapeDtypeStruct((_WORD_NUM, 128), jnp.float32),
                   jax.ShapeDtypeStruct((1, 2), jnp.float32)],
        scratch_shapes=[pltpu.SMEM((2,), jnp.float32)],
    )(wT, eT)


def _sc_body(pad, ebias, wbias,
             users_hbm, items_hbm, rev_hbm, qw_hbm, negi_hbm, negw_hbm,
             user_out, item_out, rev_out, qsum_out, ibias_out, rbias_out,
             negi_out, negw_out, negib_out, negwb_out,
             idx512, idxq, idx64, rowbuf, cbuf, bias512, sem):
    c = lax.axis_index("c")
    s = lax.axis_index("s")
    wid = s * _NC + c
    base = wid * _BC

    def extract(n, out, off, lb):
        def body(r, carry):
            cbuf[r, pl.ds(0, 16)] = rowbuf[r, pl.ds(lb, 16)]
            cbuf[r, pl.ds(16, 16)] = rowbuf[r, pl.ds(lb + 16, 16)]
            return carry
        lax.fori_loop(0, n, body, 0)
        pltpu.sync_copy(cbuf.at[pl.ds(0, n)], out.at[pl.ds(off, n)])

    def gather_rows(idxref, n):
        pltpu.async_copy(pad.at[idxref], rowbuf.at[pl.ds(0, n)], sem).wait()

    pltpu.sync_copy(users_hbm.at[pl.ds(base, _BC)], idx512)
    gather_rows(idx512, _BC)
    extract(_BC, user_out, base, _EMBED)

    pltpu.sync_copy(items_hbm.at[pl.ds(base, _BC)], idx512)
    gather_rows(idx512, _BC)
    extract(_BC, item_out, base, _EMBED)
    pltpu.async_copy(ebias.at[idx512], bias512, sem).wait()
    pltpu.sync_copy(bias512, ibias_out.at[pl.ds(base, _BC)])

    pltpu.sync_copy(rev_hbm.at[pl.ds(base, _BC)], idx512)
    gather_rows(idx512, _BC)
    extract(_BC, rev_out, base, 0)
    pltpu.async_copy(wbias.at[idx512], bias512, sem).wait()
    pltpu.sync_copy(bias512, rbias_out.at[pl.ds(base, _BC)])

    for ch in range(_QCH):
        pltpu.sync_copy(qw_hbm.at[pl.ds(wid * _QC + ch * _QROWS, _QROWS)], idxq)
        pltpu.async_copy(pad.at[idxq], rowbuf.at[pl.ds(0, _QROWS)], sem).wait()

        def gbody(g, carry):
            row = g * _L
            for h in range(2):
                sl = pl.ds(h * 16, 16)
                acc = rowbuf[row, sl]
                for l in range(1, _L):
                    acc = acc + rowbuf[row + l, sl]
                cbuf[ch * _QG + g, sl] = acc
            return carry
        lax.fori_loop(0, _QG, gbody, 0)
    pltpu.sync_copy(cbuf, qsum_out.at[pl.ds(base, _BC)])

    @pl.when(wid == 0)
    def _():
        pltpu.sync_copy(negi_hbm, idx64)
        gather_rows(idx64, _K)
        extract(_K, negi_out, 0, _EMBED)
        pltpu.async_copy(ebias.at[idx64], bias512.at[pl.ds(0, _K)], sem).wait()
        pltpu.sync_copy(bias512.at[pl.ds(0, _K)], negib_out)

        pltpu.sync_copy(negw_hbm, idx64)
        gather_rows(idx64, _K)
        extract(_K, negw_out, 0, 0)
        pltpu.async_copy(wbias.at[idx64], bias512.at[pl.ds(0, _K)], sem).wait()
        pltpu.sync_copy(bias512.at[pl.ds(0, _K)], negwb_out)


_sc_gather = pl.kernel(
    _sc_body,
    out_type=[
        jax.ShapeDtypeStruct((_B, _EMBED), jnp.float32),
        jax.ShapeDtypeStruct((_B, _EMBED), jnp.float32),
        jax.ShapeDtypeStruct((_B, _EMBED), jnp.float32),
        jax.ShapeDtypeStruct((_B, _EMBED), jnp.float32),
        jax.ShapeDtypeStruct((_B,), jnp.float32),
        jax.ShapeDtypeStruct((_B,), jnp.float32),
        jax.ShapeDtypeStruct((_K, _EMBED), jnp.float32),
        jax.ShapeDtypeStruct((_K, _EMBED), jnp.float32),
        jax.ShapeDtypeStruct((_K,), jnp.float32),
        jax.ShapeDtypeStruct((_K,), jnp.float32),
    ],
    mesh=plsc.VectorSubcoreMesh(core_axis_name="c", subcore_axis_name="s",
                                num_cores=_NC, num_subcores=_NS),
    compiler_params=pltpu.CompilerParams(use_tc_tiling_on_sc=False),
    scratch_types=[
        pltpu.VMEM((_BC,), jnp.int32),
        pltpu.VMEM((_QROWS,), jnp.int32),
        pltpu.VMEM((_K,), jnp.int32),
        pltpu.VMEM((_QROWS, 128), jnp.float32),
        pltpu.VMEM((_BC, _EMBED), jnp.float32),
        pltpu.VMEM((_BC,), jnp.float32),
        pltpu.SemaphoreType.DMA,
    ],
)


_GB = 16
_BCH = _B // _GB


def _softplus(x):
    return jnp.maximum(x, 0.0) + jnp.log1p(jnp.exp(-jnp.abs(x)))


def _k2_body(q4, u4, it4, rv4, ib, rb, wq, bq2, negi4, negw4,
             nib, nwb, nrm, o_ref, acc):
    i = pl.program_id(0)

    @pl.when(i == 0)
    def _():
        acc[0] = 0.0

    def untile(x4):
        xt = jnp.transpose(x4[...])
        return jnp.concatenate([xt[32 * k:32 * (k + 1), :] for k in range(4)],
                               axis=1)

    qT = untile(q4) * (1.0 / _L)
    uT = untile(u4)
    itT = untile(it4)
    rvT = untile(rv4)
    ngi = untile(negi4)
    ngw = untile(negw4)

    qpT = jnp.tanh(
        lax.dot_general(wq[...], qT, (((1,), (0,)), ((), ())),
                        preferred_element_type=jnp.float32) + bq2[...])
    persT = _FACTOR * qpT + (1.0 - _FACTOR) * uT

    def nll(anchorT, posT, pb, negsT, nb):
        pos_s = jnp.sum(anchorT * posT, axis=0) + pb
        neg_s = lax.dot_general(negsT, anchorT, (((0,), (0,)), ((), ())),
                                preferred_element_type=jnp.float32) + nb
        return jnp.sum(_softplus(-pos_s)) + jnp.sum(_softplus(neg_s))

    total = (nll(uT, rvT, rb[...], ngw, nwb[...])
             + nll(itT, rvT, rb[...], ngw, nwb[...])
             + nll(persT, itT, ib[...], ngi, nib[...]))
    acc[0] += total

    @pl.when(i == _GB - 1)
    def _():
        o_ref[0, 0] = acc[0] * (1.0 / _B) + _L2 * (nrm[0, 0] + nrm[0, 1])


def _k2(qsum4, user4, item4, rev4, ibias_p, rbias_p, Wq, bq2, negi4, negw4,
        nib_p, nwb_p, nrm):
    fullN = pl.BlockSpec((_K // 4, 128), lambda i: (0, 0))
    rowblk = pl.BlockSpec((_BCH // 4, 128), lambda i: (i, 0))
    return pl.pallas_call(
        _k2_body,
        grid=(_GB,),
        in_specs=[
            rowblk, rowblk, rowblk, rowblk,
            pl.BlockSpec((_BCH,), lambda i: (i,)),
            pl.BlockSpec((_BCH,), lambda i: (i,)),
            pl.BlockSpec((_EMBED, _EMBED), lambda i: (0, 0)),
            pl.BlockSpec((_EMBED, 1), lambda i: (0, 0)),
            fullN, fullN,
            pl.BlockSpec((_K, 1), lambda i: (0, 0)),
            pl.BlockSpec((_K, 1), lambda i: (0, 0)),
            pl.BlockSpec(memory_space=pltpu.SMEM),
        ],
        out_specs=pl.BlockSpec(memory_space=pltpu.SMEM),
        out_shape=jax.ShapeDtypeStruct((1, 1), jnp.float32),
        scratch_shapes=[pltpu.SMEM((1,), jnp.float32)],
    )(qsum4, user4, item4, rev4, ibias_p, rbias_p, Wq, bq2, negi4, negw4,
      nib_p, nwb_p, nrm)


def kernel(word_embedding, word_bias, entity_embedding, entity_bias, Wq, bq,
           users, items, query_words, review_words, neg_items, neg_review_words):
    i32 = lambda x: x.astype(jnp.int32)
    users1 = i32(users)
    items1 = i32(items)
    rev1 = i32(review_words)
    qw1 = i32(query_words).reshape(_B * _L)
    negi1 = i32(neg_items)
    negw1 = i32(neg_review_words)

    pad, nrm = _k1(word_embedding.T, entity_embedding.T)
    wb = word_bias.reshape(_WORD_NUM)
    eb = entity_bias.reshape(_ENTITY_NUM)

    (user_rows, item_rows, rev_rows, qsum_rows, ibias, rbias,
     negi_rows, negw_rows, negib, negwb) = _sc_gather(
        pad, eb, wb, users1, items1, rev1, qw1, negi1, negw1)

    def permB(x):
        return x.reshape(_GB, _BCH // 4, 4).transpose(0, 2, 1).reshape(_B)

    def permK(x):
        return x.reshape(_K // 4, 4).transpose(1, 0).reshape(_K, 1)

    loss = _k2(qsum_rows.reshape(-1, 128), user_rows.reshape(-1, 128),
               item_rows.reshape(-1, 128), rev_rows.reshape(-1, 128),
               permB(ibias), permB(rbias), Wq, bq.reshape(_EMBED, 1),
               negi_rows.reshape(-1, 128), negw_rows.reshape(-1, 128),
               permK(negib), permK(negwb), nrm)
    return loss.reshape(())


# speedup vs baseline: 1.3227x; 1.3227x over previous
import jax
import jax.numpy as jnp
from jax import lax
from jax.experimental import pallas as pl
from jax.experimental.pallas import tpu as pltpu
from jax.experimental.pallas import tpu_sc as plsc

_WORD_NUM = 1000000
_ENTITY_NUM = 1000000
_EMBED = 32
_FACTOR = 0.5
_L2 = 1e-06
_B = 16384
_L = 20
_K = 64

_NC, _NS = 2, 16
_NW = _NC * _NS
_BC = _B // _NW
_QC = _B * _L // _NW
_QCH = 16
_QG = 32
_QROWS = _QG * _L


_CJ = 16384
_G1 = (_WORD_NUM + _CJ - 1) // _CJ


def _k1_body(wt, et, pad, nrm, acc):
    i = pl.program_id(0)

    @pl.when(i == 0)
    def _():
        acc[0] = 0.0
        acc[1] = 0.0

    limit = _WORD_NUM - i * _CJ
    col = lax.broadcasted_iota(jnp.int32, (_EMBED, _CJ), 1)
    m = col < limit
    wr = wt[...]
    er = et[...]
    w = jnp.where(m, wr, 0.0)
    e = jnp.where(m, er, 0.0)
    acc[0] += jnp.sum(w * w)
    acc[1] += jnp.sum(e * e)
    z = jnp.zeros((128 - 2 * _EMBED, _CJ), jnp.float32)
    pad[...] = jnp.transpose(jnp.concatenate([wr, er, z], axis=0))

    @pl.when(i == _G1 - 1)
    def _():
        nrm[0, 0] = jnp.sqrt(acc[0])
        nrm[0, 1] = jnp.sqrt(acc[1])


def _k1(wT, eT):
    return pl.pallas_call(
        _k1_body,
        grid=(_G1,),
        in_specs=[pl.BlockSpec((_EMBED, _CJ), lambda i: (0, i)),
                  pl.BlockSpec((_EMBED, _CJ), lambda i: (0, i))],
        out_specs=[pl.BlockSpec((_CJ, 128), lambda i: (i, 0)),
                   pl.BlockSpec(memory_space=pltpu.SMEM)],
        out_shape=[jax.ShapeDtypeStruct((_WORD_NUM, 128), jnp.float32),
                   jax.ShapeDtypeStruct((1, 2), jnp.float32)],
        scratch_shapes=[pltpu.SMEM((2,), jnp.float32)],
    )(wT, eT)


def _sc_body(pad, ebias, wbias,
             users_hbm, items_hbm, rev_hbm, qw_hbm, negi_hbm, negw_hbm,
             user_out, item_out, rev_out, qsum_out, ibias_out, rbias_out,
             negi_out, negw_out, negib_out, negwb_out,
             idx512, idxq, idx64, rowbuf, cbuf, bias512, sem):
    c = lax.axis_index("c")
    s = lax.axis_index("s")
    wid = s * _NC + c
    base = wid * _BC

    def extract(n, out, off, lb):
        def body(r, carry):
            cbuf[r, pl.ds(0, 16)] = rowbuf[r, pl.ds(lb, 16)]
            cbuf[r, pl.ds(16, 16)] = rowbuf[r, pl.ds(lb + 16, 16)]
            return carry
        lax.fori_loop(0, n, body, 0)
        pltpu.sync_copy(cbuf.at[pl.ds(0, n)], out.at[pl.ds(off, n)])

    def gather_rows(idxref, n):
        pltpu.async_copy(pad.at[idxref], rowbuf.at[pl.ds(0, n)], sem).wait()

    pltpu.sync_copy(users_hbm.at[pl.ds(base, _BC)], idx512)
    gather_rows(idx512, _BC)
    extract(_BC, user_out, base, _EMBED)

    pltpu.sync_copy(items_hbm.at[pl.ds(base, _BC)], idx512)
    gather_rows(idx512, _BC)
    extract(_BC, item_out, base, _EMBED)
    pltpu.async_copy(ebias.at[idx512], bias512, sem).wait()
    pltpu.sync_copy(bias512, ibias_out.at[pl.ds(base, _BC)])

    pltpu.sync_copy(rev_hbm.at[pl.ds(base, _BC)], idx512)
    gather_rows(idx512, _BC)
    extract(_BC, rev_out, base, 0)
    pltpu.async_copy(wbias.at[idx512], bias512, sem).wait()
    pltpu.sync_copy(bias512, rbias_out.at[pl.ds(base, _BC)])

    for ch in range(_QCH):
        pltpu.sync_copy(qw_hbm.at[pl.ds(wid * _QC + ch * _QROWS, _QROWS)], idxq)
        pltpu.async_copy(pad.at[idxq], rowbuf.at[pl.ds(0, _QROWS)], sem).wait()

        def gbody(g, carry):
            row = g * _L
            for h in range(2):
                sl = pl.ds(h * 16, 16)
                acc = rowbuf[row, sl]
                for l in range(1, _L):
                    acc = acc + rowbuf[row + l, sl]
                cbuf[ch * _QG + g, sl] = acc
            return carry
        lax.fori_loop(0, _QG, gbody, 0)
    pltpu.sync_copy(cbuf, qsum_out.at[pl.ds(base, _BC)])

    @pl.when(wid == 0)
    def _():
        pltpu.sync_copy(negi_hbm, idx64)
        gather_rows(idx64, _K)
        extract(_K, negi_out, 0, _EMBED)
        pltpu.async_copy(ebias.at[idx64], bias512.at[pl.ds(0, _K)], sem).wait()
        pltpu.sync_copy(bias512.at[pl.ds(0, _K)], negib_out)

        pltpu.sync_copy(negw_hbm, idx64)
        gather_rows(idx64, _K)
        extract(_K, negw_out, 0, 0)
        pltpu.async_copy(wbias.at[idx64], bias512.at[pl.ds(0, _K)], sem).wait()
        pltpu.sync_copy(bias512.at[pl.ds(0, _K)], negwb_out)


_sc_gather = pl.kernel(
    _sc_body,
    out_type=[
        jax.ShapeDtypeStruct((_B, _EMBED), jnp.float32),
        jax.ShapeDtypeStruct((_B, _EMBED), jnp.float32),
        jax.ShapeDtypeStruct((_B, _EMBED), jnp.float32),
        jax.ShapeDtypeStruct((_B, _EMBED), jnp.float32),
        jax.ShapeDtypeStruct((_B,), jnp.float32),
        jax.ShapeDtypeStruct((_B,), jnp.float32),
        jax.ShapeDtypeStruct((_K, _EMBED), jnp.float32),
        jax.ShapeDtypeStruct((_K, _EMBED), jnp.float32),
        jax.ShapeDtypeStruct((_K,), jnp.float32),
        jax.ShapeDtypeStruct((_K,), jnp.float32),
    ],
    mesh=plsc.VectorSubcoreMesh(core_axis_name="c", subcore_axis_name="s",
                                num_cores=_NC, num_subcores=_NS),
    compiler_params=pltpu.CompilerParams(use_tc_tiling_on_sc=False),
    scratch_types=[
        pltpu.VMEM((_BC,), jnp.int32),
        pltpu.VMEM((_QROWS,), jnp.int32),
        pltpu.VMEM((_K,), jnp.int32),
        pltpu.VMEM((_QROWS, 128), jnp.float32),
        pltpu.VMEM((_BC, _EMBED), jnp.float32),
        pltpu.VMEM((_BC,), jnp.float32),
        pltpu.SemaphoreType.DMA,
    ],
)


_GB = 16
_BCH = _B // _GB


def _softplus(x):
    return jnp.maximum(x, 0.0) + jnp.log1p(jnp.exp(-jnp.abs(x)))


def _k2_body(q4, u4, it4, rv4, ib, rb, wq, bq2, negi4, negw4,
             nib, nwb, nrm, o_ref, acc):
    i = pl.program_id(0)

    @pl.when(i == 0)
    def _():
        acc[0] = 0.0

    def untile(x4):
        xt = jnp.transpose(x4[...])
        return jnp.concatenate([xt[32 * k:32 * (k + 1), :] for k in range(4)],
                               axis=1)

    qT = untile(q4) * (1.0 / _L)
    uT = untile(u4)
    itT = untile(it4)
    rvT = untile(rv4)
    ngi = untile(negi4)
    ngw = untile(negw4)

    qpT = jnp.tanh(
        lax.dot_general(wq[...], qT, (((1,), (0,)), ((), ())),
                        preferred_element_type=jnp.float32) + bq2[...])
    persT = _FACTOR * qpT + (1.0 - _FACTOR) * uT

    def nll(anchorT, posT, pb, negsT, nb):
        pos_s = jnp.sum(anchorT * posT, axis=0) + pb
        neg_s = lax.dot_general(negsT, anchorT, (((0,), (0,)), ((), ())),
                                preferred_element_type=jnp.float32) + nb
        return jnp.sum(_softplus(-pos_s)) + jnp.sum(_softplus(neg_s))

    total = (nll(uT, rvT, rb[...], ngw, nwb[...])
             + nll(itT, rvT, rb[...], ngw, nwb[...])
             + nll(persT, itT, ib[...], ngi, nib[...]))
    acc[0] += total

    @pl.when(i == _GB - 1)
    def _():
        o_ref[0, 0] = acc[0] * (1.0 / _B) + _L2 * (nrm[0, 0] + nrm[0, 1])


def _k2(qsum4, user4, item4, rev4, ibias_p, rbias_p, Wq, bq2, negi4, negw4,
        nib_p, nwb_p, nrm):
    fullN = pl.BlockSpec((_K // 4, 128), lambda i: (0, 0))
    rowblk = pl.BlockSpec((_BCH // 4, 128), lambda i: (i, 0))
    return pl.pallas_call(
        _k2_body,
        grid=(_GB,),
        in_specs=[
            rowblk, rowblk, rowblk, rowblk,
            pl.BlockSpec((_BCH,), lambda i: (i,)),
            pl.BlockSpec((_BCH,), lambda i: (i,)),
            pl.BlockSpec((_EMBED, _EMBED), lambda i: (0, 0)),
            pl.BlockSpec((_EMBED, 1), lambda i: (0, 0)),
            fullN, fullN,
            pl.BlockSpec((_K, 1), lambda i: (0, 0)),
            pl.BlockSpec((_K, 1), lambda i: (0, 0)),
            pl.BlockSpec(memory_space=pltpu.SMEM),
        ],
        out_specs=pl.BlockSpec(memory_space=pltpu.SMEM),
        out_shape=jax.ShapeDtypeStruct((1, 1), jnp.float32),
        scratch_shapes=[pltpu.SMEM((1,), jnp.float32)],
    )(qsum4, user4, item4, rev4, ibias_p, rbias_p, Wq, bq2, negi4, negw4,
      nib_p, nwb_p, nrm)


def kernel(word_embedding, word_bias, entity_embedding, entity_bias, Wq, bq,
           users, items, query_words, review_words, neg_items, neg_review_words):
    i32 = lambda x: x.astype(jnp.int32)
    users1 = i32(users)
    items1 = i32(items)
    rev1 = i32(review_words)
    qw1 = i32(query_words).reshape(_B * _L)
    negi1 = i32(neg_items)
    negw1 = i32(neg_review_words)

    pad, nrm = _k1(word_embedding.T, entity_embedding.T)
    wb = word_bias.reshape(_WORD_NUM)
    eb = entity_bias.reshape(_ENTITY_NUM)

    (user_rows, item_rows, rev_rows, qsum_rows, ibias, rbias,
     negi_rows, negw_rows, negib, negwb) = _sc_gather(
        pad, eb, wb, users1, items1, rev1, qw1, negi1, negw1)

    def permB(x):
        return x.reshape(_GB, _BCH // 4, 4).transpose(0, 2, 1).reshape(_B)

    def permK(x):
        return x.reshape(_K // 4, 4).transpose(1, 0).reshape(_K, 1)

    loss = _k2(qsum_rows.reshape(-1, 128), user_rows.reshape(-1, 128),
               item_rows.reshape(-1, 128), rev_rows.reshape(-1, 128),
               permB(ibias), permB(rbias), Wq, bq.reshape(_EMBED, 1),
               negi_rows.reshape(-1, 128), negw_rows.reshape(-1, 128),
               permK(negib), permK(negwb), nrm)
    return loss.reshape(())


# speedup vs baseline: 1.3378x; 1.0114x over previous
import jax
import jax.numpy as jnp
from jax import lax
from jax.experimental import pallas as pl
from jax.experimental.pallas import tpu as pltpu
from jax.experimental.pallas import tpu_sc as plsc

_WORD_NUM = 1000000
_ENTITY_NUM = 1000000
_EMBED = 32
_FACTOR = 0.5
_L2 = 1e-06
_B = 16384
_L = 20
_K = 64

_NC, _NS = 2, 16
_NW = _NC * _NS
_BC = _B // _NW
_QC = _B * _L // _NW
_QCH = 16
_QG = 32
_QROWS = _QG * _L


_CJ = 32768
_G1 = (_WORD_NUM + _CJ - 1) // _CJ


def _k1_body(wt, et, pad, nrm, acc):
    i = pl.program_id(0)

    @pl.when(i == 0)
    def _():
        acc[0] = 0.0
        acc[1] = 0.0

    limit = _WORD_NUM - i * _CJ
    col = lax.broadcasted_iota(jnp.int32, (_EMBED, _CJ), 1)
    m = col < limit
    wr = wt[...]
    er = et[...]
    w = jnp.where(m, wr, 0.0)
    e = jnp.where(m, er, 0.0)
    acc[0] += jnp.sum(w * w)
    acc[1] += jnp.sum(e * e)
    z = jnp.zeros((128 - 2 * _EMBED, _CJ), jnp.float32)
    pad[...] = jnp.transpose(jnp.concatenate([wr, er, z], axis=0))

    @pl.when(i == _G1 - 1)
    def _():
        nrm[0, 0] = jnp.sqrt(acc[0])
        nrm[0, 1] = jnp.sqrt(acc[1])


def _k1(wT, eT):
    return pl.pallas_call(
        _k1_body,
        grid=(_G1,),
        in_specs=[pl.BlockSpec((_EMBED, _CJ), lambda i: (0, i)),
                  pl.BlockSpec((_EMBED, _CJ), lambda i: (0, i))],
        out_specs=[pl.BlockSpec((_CJ, 128), lambda i: (i, 0)),
                   pl.BlockSpec(memory_space=pltpu.SMEM)],
        out_shape=[jax.ShapeDtypeStruct((_WORD_NUM, 128), jnp.float32),
                   jax.ShapeDtypeStruct((1, 2), jnp.float32)],
        scratch_shapes=[pltpu.SMEM((2,), jnp.float32)],
    )(wT, eT)


def _sc_body(pad, ebias, wbias,
             users_hbm, items_hbm, rev_hbm, qw_hbm, negi_hbm, negw_hbm,
             user_out, item_out, rev_out, qsum_out, ibias_out, rbias_out,
             negi_out, negw_out, negib_out, negwb_out,
             idx512, idxq, idx64, rowbuf, cbuf, bias512, sem):
    c = lax.axis_index("c")
    s = lax.axis_index("s")
    wid = s * _NC + c
    base = wid * _BC

    def extract(n, out, off, lb):
        def body(r, carry):
            cbuf[r, pl.ds(0, 16)] = rowbuf[r, pl.ds(lb, 16)]
            cbuf[r, pl.ds(16, 16)] = rowbuf[r, pl.ds(lb + 16, 16)]
            return carry
        lax.fori_loop(0, n, body, 0)
        pltpu.sync_copy(cbuf.at[pl.ds(0, n)], out.at[pl.ds(off, n)])

    def gather_rows(idxref, n):
        pltpu.async_copy(pad.at[idxref], rowbuf.at[pl.ds(0, n)], sem).wait()

    pltpu.sync_copy(users_hbm.at[pl.ds(base, _BC)], idx512)
    gather_rows(idx512, _BC)
    extract(_BC, user_out, base, _EMBED)

    pltpu.sync_copy(items_hbm.at[pl.ds(base, _BC)], idx512)
    gather_rows(idx512, _BC)
    extract(_BC, item_out, base, _EMBED)
    pltpu.async_copy(ebias.at[idx512], bias512, sem).wait()
    pltpu.sync_copy(bias512, ibias_out.at[pl.ds(base, _BC)])

    pltpu.sync_copy(rev_hbm.at[pl.ds(base, _BC)], idx512)
    gather_rows(idx512, _BC)
    extract(_BC, rev_out, base, 0)
    pltpu.async_copy(wbias.at[idx512], bias512, sem).wait()
    pltpu.sync_copy(bias512, rbias_out.at[pl.ds(base, _BC)])

    for ch in range(_QCH):
        pltpu.sync_copy(qw_hbm.at[pl.ds(wid * _QC + ch * _QROWS, _QROWS)], idxq)
        pltpu.async_copy(pad.at[idxq], rowbuf.at[pl.ds(0, _QROWS)], sem).wait()

        def gbody(g, carry):
            row = g * _L
            for h in range(2):
                sl = pl.ds(h * 16, 16)
                acc = rowbuf[row, sl]
                for l in range(1, _L):
                    acc = acc + rowbuf[row + l, sl]
                cbuf[ch * _QG + g, sl] = acc
            return carry
        lax.fori_loop(0, _QG, gbody, 0)
    pltpu.sync_copy(cbuf, qsum_out.at[pl.ds(base, _BC)])

    @pl.when(wid == 0)
    def _():
        pltpu.sync_copy(negi_hbm, idx64)
        gather_rows(idx64, _K)
        extract(_K, negi_out, 0, _EMBED)
        pltpu.async_copy(ebias.at[idx64], bias512.at[pl.ds(0, _K)], sem).wait()
        pltpu.sync_copy(bias512.at[pl.ds(0, _K)], negib_out)

        pltpu.sync_copy(negw_hbm, idx64)
        gather_rows(idx64, _K)
        extract(_K, negw_out, 0, 0)
        pltpu.async_copy(wbias.at[idx64], bias512.at[pl.ds(0, _K)], sem).wait()
        pltpu.sync_copy(bias512.at[pl.ds(0, _K)], negwb_out)


_sc_gather = pl.kernel(
    _sc_body,
    out_type=[
        jax.ShapeDtypeStruct((_B, _EMBED), jnp.float32),
        jax.ShapeDtypeStruct((_B, _EMBED), jnp.float32),
        jax.ShapeDtypeStruct((_B, _EMBED), jnp.float32),
        jax.ShapeDtypeStruct((_B, _EMBED), jnp.float32),
        jax.ShapeDtypeStruct((_B,), jnp.float32),
        jax.ShapeDtypeStruct((_B,), jnp.float32),
        jax.ShapeDtypeStruct((_K, _EMBED), jnp.float32),
        jax.ShapeDtypeStruct((_K, _EMBED), jnp.float32),
        jax.ShapeDtypeStruct((_K,), jnp.float32),
        jax.ShapeDtypeStruct((_K,), jnp.float32),
    ],
    mesh=plsc.VectorSubcoreMesh(core_axis_name="c", subcore_axis_name="s",
                                num_cores=_NC, num_subcores=_NS),
    compiler_params=pltpu.CompilerParams(use_tc_tiling_on_sc=False),
    scratch_types=[
        pltpu.VMEM((_BC,), jnp.int32),
        pltpu.VMEM((_QROWS,), jnp.int32),
        pltpu.VMEM((_K,), jnp.int32),
        pltpu.VMEM((_QROWS, 128), jnp.float32),
        pltpu.VMEM((_BC, _EMBED), jnp.float32),
        pltpu.VMEM((_BC,), jnp.float32),
        pltpu.SemaphoreType.DMA,
    ],
)


_GB = 16
_BCH = _B // _GB


def _softplus(x):
    return jnp.maximum(x, 0.0) + jnp.log1p(jnp.exp(-jnp.abs(x)))


def _k2_body(q4, u4, it4, rv4, ib, rb, wq, bq2, negi4, negw4,
             nib, nwb, nrm, o_ref, acc):
    i = pl.program_id(0)

    @pl.when(i == 0)
    def _():
        acc[0] = 0.0

    def untile(x4):
        xt = jnp.transpose(x4[...])
        return jnp.concatenate([xt[32 * k:32 * (k + 1), :] for k in range(4)],
                               axis=1)

    qT = untile(q4) * (1.0 / _L)
    uT = untile(u4)
    itT = untile(it4)
    rvT = untile(rv4)
    ngi = untile(negi4)
    ngw = untile(negw4)

    qpT = jnp.tanh(
        lax.dot_general(wq[...], qT, (((1,), (0,)), ((), ())),
                        preferred_element_type=jnp.float32) + bq2[...])
    persT = _FACTOR * qpT + (1.0 - _FACTOR) * uT

    def nll(anchorT, posT, pb, negsT, nb):
        pos_s = jnp.sum(anchorT * posT, axis=0) + pb
        neg_s = lax.dot_general(negsT, anchorT, (((0,), (0,)), ((), ())),
                                preferred_element_type=jnp.float32) + nb
        return jnp.sum(_softplus(-pos_s)) + jnp.sum(_softplus(neg_s))

    total = (nll(uT, rvT, rb[...], ngw, nwb[...])
             + nll(itT, rvT, rb[...], ngw, nwb[...])
             + nll(persT, itT, ib[...], ngi, nib[...]))
    acc[0] += total

    @pl.when(i == _GB - 1)
    def _():
        o_ref[0, 0] = acc[0] * (1.0 / _B) + _L2 * (nrm[0, 0] + nrm[0, 1])


def _k2(qsum4, user4, item4, rev4, ibias_p, rbias_p, Wq, bq2, negi4, negw4,
        nib_p, nwb_p, nrm):
    fullN = pl.BlockSpec((_K // 4, 128), lambda i: (0, 0))
    rowblk = pl.BlockSpec((_BCH // 4, 128), lambda i: (i, 0))
    return pl.pallas_call(
        _k2_body,
        grid=(_GB,),
        in_specs=[
            rowblk, rowblk, rowblk, rowblk,
            pl.BlockSpec((_BCH,), lambda i: (i,)),
            pl.BlockSpec((_BCH,), lambda i: (i,)),
            pl.BlockSpec((_EMBED, _EMBED), lambda i: (0, 0)),
            pl.BlockSpec((_EMBED, 1), lambda i: (0, 0)),
            fullN, fullN,
            pl.BlockSpec((_K, 1), lambda i: (0, 0)),
            pl.BlockSpec((_K, 1), lambda i: (0, 0)),
            pl.BlockSpec(memory_space=pltpu.SMEM),
        ],
        out_specs=pl.BlockSpec(memory_space=pltpu.SMEM),
        out_shape=jax.ShapeDtypeStruct((1, 1), jnp.float32),
        scratch_shapes=[pltpu.SMEM((1,), jnp.float32)],
    )(qsum4, user4, item4, rev4, ibias_p, rbias_p, Wq, bq2, negi4, negw4,
      nib_p, nwb_p, nrm)


def kernel(word_embedding, word_bias, entity_embedding, entity_bias, Wq, bq,
           users, items, query_words, review_words, neg_items, neg_review_words):
    i32 = lambda x: x.astype(jnp.int32)
    users1 = i32(users)
    items1 = i32(items)
    rev1 = i32(review_words)
    qw1 = i32(query_words).reshape(_B * _L)
    negi1 = i32(neg_items)
    negw1 = i32(neg_review_words)

    pad, nrm = _k1(word_embedding.T, entity_embedding.T)
    wb = word_bias.reshape(_WORD_NUM)
    eb = entity_bias.reshape(_ENTITY_NUM)

    (user_rows, item_rows, rev_rows, qsum_rows, ibias, rbias,
     negi_rows, negw_rows, negib, negwb) = _sc_gather(
        pad, eb, wb, users1, items1, rev1, qw1, negi1, negw1)

    def permB(x):
        return x.reshape(_GB, _BCH // 4, 4).transpose(0, 2, 1).reshape(_B)

    def permK(x):
        return x.reshape(_K // 4, 4).transpose(1, 0).reshape(_K, 1)

    loss = _k2(qsum_rows.reshape(-1, 128), user_rows.reshape(-1, 128),
               item_rows.reshape(-1, 128), rev_rows.reshape(-1, 128),
               permB(ibias), permB(rbias), Wq, bq.reshape(_EMBED, 1),
               negi_rows.reshape(-1, 128), negw_rows.reshape(-1, 128),
               permK(negib), permK(negwb), nrm)
    return loss.reshape(())


# speedup vs baseline: 1.4231x; 1.0638x over previous
import jax
import jax.numpy as jnp
from jax import lax
from jax.experimental import pallas as pl
from jax.experimental.pallas import tpu as pltpu
from jax.experimental.pallas import tpu_sc as plsc

_WORD_NUM = 1000000
_ENTITY_NUM = 1000000
_EMBED = 32
_FACTOR = 0.5
_L2 = 1e-06
_B = 16384
_L = 20
_K = 64

_NC, _NS = 2, 16
_NW = _NC * _NS
_BC = _B // _NW
_QC = _B * _L // _NW
_QCH = 32
_QG = 16
_QROWS = _QG * _L


_CJ = 32768
_G1 = (_WORD_NUM + _CJ - 1) // _CJ


def _k1_body(wt, et, pad, nrm, acc):
    i = pl.program_id(0)

    @pl.when(i == 0)
    def _():
        acc[0] = 0.0
        acc[1] = 0.0

    limit = _WORD_NUM - i * _CJ
    col = lax.broadcasted_iota(jnp.int32, (_EMBED, _CJ), 1)
    m = col < limit
    wr = wt[...]
    er = et[...]
    w = jnp.where(m, wr, 0.0)
    e = jnp.where(m, er, 0.0)
    acc[0] += jnp.sum(w * w)
    acc[1] += jnp.sum(e * e)
    z = jnp.zeros((128 - 2 * _EMBED, _CJ), jnp.float32)
    pad[...] = jnp.transpose(jnp.concatenate([wr, er, z], axis=0))

    @pl.when(i == _G1 - 1)
    def _():
        nrm[0, 0] = jnp.sqrt(acc[0])
        nrm[0, 1] = jnp.sqrt(acc[1])


def _k1(wT, eT):
    return pl.pallas_call(
        _k1_body,
        grid=(_G1,),
        in_specs=[pl.BlockSpec((_EMBED, _CJ), lambda i: (0, i)),
                  pl.BlockSpec((_EMBED, _CJ), lambda i: (0, i))],
        out_specs=[pl.BlockSpec((_CJ, 128), lambda i: (i, 0)),
                   pl.BlockSpec(memory_space=pltpu.SMEM)],
        out_shape=[jax.ShapeDtypeStruct((_WORD_NUM, 128), jnp.float32),
                   jax.ShapeDtypeStruct((1, 2), jnp.float32)],
        scratch_shapes=[pltpu.SMEM((2,), jnp.float32)],
    )(wT, eT)


def _sc_body(pad, ebias, wbias,
             users_hbm, items_hbm, rev_hbm, qw_hbm, negi_hbm, negw_hbm,
             user_out, item_out, rev_out, qsum_out, ibias_out, rbias_out,
             negi_out, negw_out, negib_out, negwb_out,
             idx512, idxqA, idxqB, idx64, rowbuf, rowbufB, cbuf, bias512, sem):
    c = lax.axis_index("c")
    s = lax.axis_index("s")
    wid = s * _NC + c
    base = wid * _BC

    def extract(n, out, off, lb):
        def body(r, carry):
            cbuf[r, pl.ds(0, 16)] = rowbuf[r, pl.ds(lb, 16)]
            cbuf[r, pl.ds(16, 16)] = rowbuf[r, pl.ds(lb + 16, 16)]
            return carry
        lax.fori_loop(0, n, body, 0)
        pltpu.sync_copy(cbuf.at[pl.ds(0, n)], out.at[pl.ds(off, n)])

    def gather_rows(idxref, n):
        pltpu.async_copy(pad.at[idxref], rowbuf.at[pl.ds(0, n)], sem).wait()

    pltpu.sync_copy(users_hbm.at[pl.ds(base, _BC)], idx512)
    gather_rows(idx512, _BC)
    extract(_BC, user_out, base, _EMBED)

    pltpu.sync_copy(items_hbm.at[pl.ds(base, _BC)], idx512)
    gather_rows(idx512, _BC)
    extract(_BC, item_out, base, _EMBED)
    pltpu.async_copy(ebias.at[idx512], bias512, sem).wait()
    pltpu.sync_copy(bias512, ibias_out.at[pl.ds(base, _BC)])

    pltpu.sync_copy(rev_hbm.at[pl.ds(base, _BC)], idx512)
    gather_rows(idx512, _BC)
    extract(_BC, rev_out, base, 0)
    pltpu.async_copy(wbias.at[idx512], bias512, sem).wait()
    pltpu.sync_copy(bias512, rbias_out.at[pl.ds(base, _BC)])

    idxbufs = (idxqA, idxqB)
    qbufs = (rowbuf, rowbufB)

    def fire(ch):
        ib = idxbufs[ch % 2]
        rb = qbufs[ch % 2]
        pltpu.sync_copy(qw_hbm.at[pl.ds(wid * _QC + ch * _QROWS, _QROWS)], ib)
        return pltpu.async_copy(pad.at[ib], rb.at[pl.ds(0, _QROWS)], sem)

    cp = fire(0)
    for ch in range(_QCH):
        nxt = fire(ch + 1) if ch + 1 < _QCH else None
        cp.wait()
        rb = qbufs[ch % 2]

        def gbody(g, carry, rb=rb, ch=ch):
            row = g * _L
            for h in range(2):
                sl = pl.ds(h * 16, 16)
                acc = rb[row, sl]
                for l in range(1, _L):
                    acc = acc + rb[row + l, sl]
                cbuf[ch * _QG + g, sl] = acc
            return carry
        lax.fori_loop(0, _QG, gbody, 0)
        cp = nxt
    pltpu.sync_copy(cbuf, qsum_out.at[pl.ds(base, _BC)])

    @pl.when(wid == 0)
    def _():
        pltpu.sync_copy(negi_hbm, idx64)
        gather_rows(idx64, _K)
        extract(_K, negi_out, 0, _EMBED)
        pltpu.async_copy(ebias.at[idx64], bias512.at[pl.ds(0, _K)], sem).wait()
        pltpu.sync_copy(bias512.at[pl.ds(0, _K)], negib_out)

        pltpu.sync_copy(negw_hbm, idx64)
        gather_rows(idx64, _K)
        extract(_K, negw_out, 0, 0)
        pltpu.async_copy(wbias.at[idx64], bias512.at[pl.ds(0, _K)], sem).wait()
        pltpu.sync_copy(bias512.at[pl.ds(0, _K)], negwb_out)


_sc_gather = pl.kernel(
    _sc_body,
    out_type=[
        jax.ShapeDtypeStruct((_B, _EMBED), jnp.float32),
        jax.ShapeDtypeStruct((_B, _EMBED), jnp.float32),
        jax.ShapeDtypeStruct((_B, _EMBED), jnp.float32),
        jax.ShapeDtypeStruct((_B, _EMBED), jnp.float32),
        jax.ShapeDtypeStruct((_B,), jnp.float32),
        jax.ShapeDtypeStruct((_B,), jnp.float32),
        jax.ShapeDtypeStruct((_K, _EMBED), jnp.float32),
        jax.ShapeDtypeStruct((_K, _EMBED), jnp.float32),
        jax.ShapeDtypeStruct((_K,), jnp.float32),
        jax.ShapeDtypeStruct((_K,), jnp.float32),
    ],
    mesh=plsc.VectorSubcoreMesh(core_axis_name="c", subcore_axis_name="s",
                                num_cores=_NC, num_subcores=_NS),
    compiler_params=pltpu.CompilerParams(use_tc_tiling_on_sc=False),
    scratch_types=[
        pltpu.VMEM((_BC,), jnp.int32),
        pltpu.VMEM((_QROWS,), jnp.int32),
        pltpu.VMEM((_QROWS,), jnp.int32),
        pltpu.VMEM((_K,), jnp.int32),
        pltpu.VMEM((_BC, 128), jnp.float32),
        pltpu.VMEM((_QROWS, 128), jnp.float32),
        pltpu.VMEM((_BC, _EMBED), jnp.float32),
        pltpu.VMEM((_BC,), jnp.float32),
        pltpu.SemaphoreType.DMA,
    ],
)


_GB = 16
_BCH = _B // _GB


def _softplus(x):
    return jnp.maximum(x, 0.0) + jnp.log1p(jnp.exp(-jnp.abs(x)))


def _k2_body(q4, u4, it4, rv4, ib, rb, wq, bq2, negi4, negw4,
             nib, nwb, nrm, o_ref, acc):
    i = pl.program_id(0)

    @pl.when(i == 0)
    def _():
        acc[0] = 0.0

    def untile(x4):
        xt = jnp.transpose(x4[...])
        return jnp.concatenate([xt[32 * k:32 * (k + 1), :] for k in range(4)],
                               axis=1)

    qT = untile(q4) * (1.0 / _L)
    uT = untile(u4)
    itT = untile(it4)
    rvT = untile(rv4)
    ngi = untile(negi4)
    ngw = untile(negw4)

    qpT = jnp.tanh(
        lax.dot_general(wq[...], qT, (((1,), (0,)), ((), ())),
                        preferred_element_type=jnp.float32) + bq2[...])
    persT = _FACTOR * qpT + (1.0 - _FACTOR) * uT

    def nll(anchorT, posT, pb, negsT, nb):
        pos_s = jnp.sum(anchorT * posT, axis=0) + pb
        neg_s = lax.dot_general(negsT, anchorT, (((0,), (0,)), ((), ())),
                                preferred_element_type=jnp.float32) + nb
        return jnp.sum(_softplus(-pos_s)) + jnp.sum(_softplus(neg_s))

    total = (nll(uT, rvT, rb[...], ngw, nwb[...])
             + nll(itT, rvT, rb[...], ngw, nwb[...])
             + nll(persT, itT, ib[...], ngi, nib[...]))
    acc[0] += total

    @pl.when(i == _GB - 1)
    def _():
        o_ref[0, 0] = acc[0] * (1.0 / _B) + _L2 * (nrm[0, 0] + nrm[0, 1])


def _k2(qsum4, user4, item4, rev4, ibias_p, rbias_p, Wq, bq2, negi4, negw4,
        nib_p, nwb_p, nrm):
    fullN = pl.BlockSpec((_K // 4, 128), lambda i: (0, 0))
    rowblk = pl.BlockSpec((_BCH // 4, 128), lambda i: (i, 0))
    return pl.pallas_call(
        _k2_body,
        grid=(_GB,),
        in_specs=[
            rowblk, rowblk, rowblk, rowblk,
            pl.BlockSpec((_BCH,), lambda i: (i,)),
            pl.BlockSpec((_BCH,), lambda i: (i,)),
            pl.BlockSpec((_EMBED, _EMBED), lambda i: (0, 0)),
            pl.BlockSpec((_EMBED, 1), lambda i: (0, 0)),
            fullN, fullN,
            pl.BlockSpec((_K, 1), lambda i: (0, 0)),
            pl.BlockSpec((_K, 1), lambda i: (0, 0)),
            pl.BlockSpec(memory_space=pltpu.SMEM),
        ],
        out_specs=pl.BlockSpec(memory_space=pltpu.SMEM),
        out_shape=jax.ShapeDtypeStruct((1, 1), jnp.float32),
        scratch_shapes=[pltpu.SMEM((1,), jnp.float32)],
    )(qsum4, user4, item4, rev4, ibias_p, rbias_p, Wq, bq2, negi4, negw4,
      nib_p, nwb_p, nrm)


def kernel(word_embedding, word_bias, entity_embedding, entity_bias, Wq, bq,
           users, items, query_words, review_words, neg_items, neg_review_words):
    i32 = lambda x: x.astype(jnp.int32)
    users1 = i32(users)
    items1 = i32(items)
    rev1 = i32(review_words)
    qw1 = i32(query_words).reshape(_B * _L)
    negi1 = i32(neg_items)
    negw1 = i32(neg_review_words)

    pad, nrm = _k1(word_embedding.T, entity_embedding.T)
    wb = word_bias.reshape(_WORD_NUM)
    eb = entity_bias.reshape(_ENTITY_NUM)

    (user_rows, item_rows, rev_rows, qsum_rows, ibias, rbias,
     negi_rows, negw_rows, negib, negwb) = _sc_gather(
        pad, eb, wb, users1, items1, rev1, qw1, negi1, negw1)

    def permB(x):
        return x.reshape(_GB, _BCH // 4, 4).transpose(0, 2, 1).reshape(_B)

    def permK(x):
        return x.reshape(_K // 4, 4).transpose(1, 0).reshape(_K, 1)

    loss = _k2(qsum_rows.reshape(-1, 128), user_rows.reshape(-1, 128),
               item_rows.reshape(-1, 128), rev_rows.reshape(-1, 128),
               permB(ibias), permB(rbias), Wq, bq.reshape(_EMBED, 1),
               negi_rows.reshape(-1, 128), negw_rows.reshape(-1, 128),
               permK(negib), permK(negwb), nrm)
    return loss.reshape(())
